# baseline (speedup 1.0000x reference)
import functools

import jax
import jax.numpy as jnp
import numpy as np
from jax import lax
from jax.experimental import pallas as pl
from jax.experimental.pallas import tpu as pltpu
from jax.experimental.pallas import tpu_sc as plsc

B = 16384
L = 12
VOCAB = 10000
D = 64

_INFO = plsc.get_sparse_core_info()
_NC = _INFO.num_cores
_NS = _INFO.num_subcores
_NW = _NC * _NS
_SEQ_PER_W = B // _NW
_IDX_PER_W = _SEQ_PER_W * L
_CHUNK_SEQ = 32
_CHUNK_IDX = _CHUNK_SEQ * L
_N_CHUNKS = _SEQ_PER_W // _CHUNK_SEQ
_STREAMS = _CHUNK_IDX // 128
_HALF = B // 2

_PERM = np.empty((D,), dtype=np.int32)
for _h in range(D // 32):
    for _i in range(16):
        _PERM[32 * _h + 2 * _i] = 32 * _h + _i
        _PERM[32 * _h + 2 * _i + 1] = 32 * _h + 16 + _i


def _table_body(e_ref, w_ref, out_ref):
    prod = lax.dot_general(
        e_ref[...], w_ref[...],
        dimension_numbers=(((1,), (1,)), ((), ())),
        preferred_element_type=jnp.float32,
    )
    out_ref[...] = (prod * (1.0 / L)).astype(jnp.bfloat16)


def _transform_table(emb_table, W):
    return pl.pallas_call(
        _table_body,
        out_shape=jax.ShapeDtypeStruct((VOCAB, D), jnp.bfloat16),
    )(emb_table, W[_PERM])


def _sc_body(table_hbm, idx_hbm, out_hbm, idx_v, rows0, rows1, out_v,
             sem0, sem1):
    wid = lax.axis_index("s") * _NC + lax.axis_index("c")
    base_idx = wid * _IDX_PER_W

    pltpu.sync_copy(idx_hbm.at[pl.ds(base_idx, _IDX_PER_W)], idx_v)

    bufs = (rows0, rows1)
    sems = (sem0, sem1)
    lo_mask = jnp.full((16,), -65536, dtype=jnp.int32)

    def copies(c, k):
        buf, sem = bufs[k], sems[k]
        return [
            pltpu.make_async_copy(
                table_hbm.at[idx_v.at[pl.ds(c * _CHUNK_IDX + j * 128, 128)]],
                buf.at[pl.ds(j * 128, 128)],
                sem,
            )
            for j in range(_STREAMS)
        ]

    def fire(c, k):
        for h in copies(c, k):
            h.start()

    def drain(c, k):
        for h in copies(c, k):
            h.wait()

    def compute(c, k):
        buf = bufs[k]

        def seq_body(s, _):
            base = s * L
            accs = [None] * 4
            for l in range(L):
                for h in range(2):
                    w = plsc.bitcast(buf[base + l, pl.ds(32 * h, 32)],
                                     jnp.int32)
                    lo = plsc.bitcast(w << 16, jnp.float32)
                    hi = plsc.bitcast(w & lo_mask, jnp.float32)
                    if l == 0:
                        accs[2 * h] = lo
                        accs[2 * h + 1] = hi
                    else:
                        accs[2 * h] = accs[2 * h] + lo
                        accs[2 * h + 1] = accs[2 * h + 1] + hi
            for q in range(4):
                out_v[c * _CHUNK_SEQ + s, pl.ds(q * 16, 16)] = accs[q]
            return 0

        lax.fori_loop(0, _CHUNK_SEQ, seq_body, 0, unroll=2)

    fire(0, 0)

    def pair_body(p):
        c0 = p * 2
        c1 = c0 + 1
        fire(c1, 1)
        drain(c0, 0)
        compute(c0, 0)

        @pl.when(c1 + 1 < _N_CHUNKS)
        def _():
            fire(c1 + 1, 0)

        drain(c1, 1)
        compute(c1, 1)

    pl.loop(0, _N_CHUNKS // 2)(pair_body)

    half = wid // _NS
    row0 = (wid % _NS) * _SEQ_PER_W
    pltpu.sync_copy(
        out_v,
        out_hbm.at[pl.ds(row0, _SEQ_PER_W), pl.ds(half * D, D)],
    )


@functools.partial(
    pl.kernel,
    out_type=jax.ShapeDtypeStruct((_HALF, 2 * D), jnp.float32),
    mesh=plsc.VectorSubcoreMesh(core_axis_name="c", subcore_axis_name="s"),
    compiler_params=pltpu.CompilerParams(use_tc_tiling_on_sc=False,
                                         needs_layout_passes=False),
    scratch_types=[
        pltpu.VMEM((_IDX_PER_W,), jnp.int32),
        pltpu.VMEM((_CHUNK_IDX, D), jnp.bfloat16),
        pltpu.VMEM((_CHUNK_IDX, D), jnp.bfloat16),
        pltpu.VMEM((_SEQ_PER_W, D), jnp.float32),
        pltpu.SemaphoreType.DMA,
        pltpu.SemaphoreType.DMA,
    ],
)
def _sc_gather_pool(table_hbm, idx_hbm, out_hbm, idx_v, rows0, rows1, out_v,
                    sem0, sem1):
    _sc_body(table_hbm, idx_hbm, out_hbm, idx_v, rows0, rows1, out_v,
             sem0, sem1)


_EPI_ROWS = 1024
_K8 = _HALF // _EPI_ROWS


def _epi_body(i_ref, b_ref, o_ref):
    half = pl.program_id(0) // _K8
    x = i_ref[...]
    o_ref[...] = jnp.where(half == 0, x[:, :D], x[:, D:]) + b_ref[...]


def _epilogue(packed, b):
    return pl.pallas_call(
        _epi_body,
        grid=(B // _EPI_ROWS,),
        in_specs=[
            pl.BlockSpec((_EPI_ROWS, 2 * D), lambda j: (j % _K8, 0)),
            pl.BlockSpec((1, D), lambda j: (0, 0)),
        ],
        out_specs=pl.BlockSpec((_EPI_ROWS, D), lambda j: (j, 0)),
        out_shape=jax.ShapeDtypeStruct((B, D), jnp.float32),
    )(packed, b.reshape(1, D))


@jax.jit
def kernel(input_ids, emb_table, W, b):
    table2 = _transform_table(emb_table, W)
    idx = input_ids.astype(jnp.int32).reshape(-1)
    packed = _sc_gather_pool(table2, idx)
    return _epilogue(packed, b)
